# Initial kernel scaffold; baseline (speedup 1.0000x reference)
import jax
import jax.numpy as jnp
from jax.experimental import pallas as pl


def kernel(node_features, adj_indices, adj_values, W1, b1, g1, be1, W2, b2, g2, be2, Wf, bf):
    raise NotImplementedError("write your pallas kernel here")


# speedup vs baseline: 4.9402x; 4.9402x over previous
import functools
import math

import jax
import jax.numpy as jnp
from jax import lax
from jax.experimental import pallas as pl
from jax.experimental.pallas import tpu as pltpu
from jax.experimental.pallas import tpu_sc as plsc

N = 10000
E = 320000
D = 128
H = 128

C = 128
NCHUNK = E // C


def _sc_aggregate(x, dst2d, src2d, val2d):
    info = plsc.get_sparse_core_info()
    NC, NS = info.num_cores, info.num_subcores
    NW = NC * NS
    rows_per_tile = (N // NS) // 8 * 8
    rem_rows = N - NS * rows_per_tile
    rem_base = NS * rows_per_tile

    mesh = plsc.VectorSubcoreMesh(core_axis_name="c", subcore_axis_name="s")

    @functools.partial(
        pl.kernel,
        mesh=mesh,
        out_type=(
            jax.ShapeDtypeStruct((N, D), jnp.float32),
            jax.ShapeDtypeStruct((N, D), jnp.float32),
        ),
        scratch_types=[
            pltpu.VMEM((1, C), jnp.int32),
            pltpu.VMEM((1, C), jnp.int32),
            pltpu.VMEM((1, C), jnp.float32),
            pltpu.VMEM((C, D), jnp.float32),
            pltpu.VMEM_SHARED((N, D), jnp.float32),
            pltpu.SemaphoreType.DMA,
        ],
    )
    def agg_kernel(x_hbm, dst_hbm, src_hbm, val_hbm, out0, out1,
                   dst_v, src_v, val_v, rows_v, acc_sh, sem):
        cid = lax.axis_index("c")
        sid = lax.axis_index("s")
        wid = sid * NC + cid

        def zrow(r, _):
            for k8 in range(D // 16):
                rows_v[r, pl.ds(16 * k8, 16)] = jnp.zeros((16,), jnp.float32)
            return 0
        lax.fori_loop(0, C, zrow, 0)
        base = sid * rows_per_tile
        tail = rows_per_tile - 4 * C
        for i in range(4):
            pltpu.sync_copy(rows_v, acc_sh.at[pl.ds(base + i * C, C)])
        pltpu.sync_copy(rows_v.at[pl.ds(0, tail)],
                        acc_sh.at[pl.ds(base + 4 * C, tail)])

        @pl.when(sid == 0)
        def _():
            pltpu.sync_copy(rows_v.at[pl.ds(0, rem_rows)],
                            acc_sh.at[pl.ds(rem_base, rem_rows)])
        plsc.subcore_barrier()

        g_lo = wid * NCHUNK // NW
        g_hi = (wid + 1) * NCHUNK // NW

        def body(g, _):
            pltpu.sync_copy(dst_hbm.at[g], dst_v)
            pltpu.sync_copy(src_hbm.at[g], src_v)
            pltpu.sync_copy(val_hbm.at[g], val_v)
            pltpu.async_copy(x_hbm.at[src_v.at[0]], rows_v, sem).wait()

            def scale(gg, _):
                val16 = val_v[0, pl.ds(gg * 16, 16)]
                for l in range(16):
                    v = val16[l]
                    j = gg * 16 + l
                    for k8 in range(D // 16):
                        sl = pl.ds(16 * k8, 16)
                        rows_v[j, sl] = rows_v[j, sl] * v
                return 0
            lax.fori_loop(0, C // 16, scale, 0)

            pltpu.sync_copy(rows_v, acc_sh.at[dst_v.at[0]], add=True)
            return 0
        lax.fori_loop(g_lo, g_hi, body, 0)
        plsc.subcore_barrier()

        def copy_out(out_ref):
            for i in range(4):
                pltpu.sync_copy(acc_sh.at[pl.ds(base + i * C, C)],
                                out_ref.at[pl.ds(base + i * C, C)])
            pltpu.sync_copy(acc_sh.at[pl.ds(base + 4 * C, tail)],
                            out_ref.at[pl.ds(base + 4 * C, tail)])

            @pl.when(sid == 0)
            def _():
                pltpu.sync_copy(acc_sh.at[pl.ds(rem_base, rem_rows)],
                                out_ref.at[pl.ds(rem_base, rem_rows)])

        @pl.when(cid == 0)
        def _():
            copy_out(out0)

        @pl.when(cid == 1)
        def _():
            copy_out(out1)

    return agg_kernel(x, dst2d, src2d, val2d)


_BR = 1000
_INV_SQRT2 = 1.0 / math.sqrt(2.0)


def _ln_gelu(h, g, be):
    mu = jnp.mean(h, axis=-1, keepdims=True)
    var = jnp.mean((h - mu) ** 2, axis=-1, keepdims=True)
    h = (h - mu) / jnp.sqrt(var + 1e-5) * g + be
    return 0.5 * h * (1.0 + lax.erf(h * _INV_SQRT2))


def _dense1_body(x_ref, a0_ref, a1_ref, W_ref, b_ref, g_ref, be_ref, o_ref):
    h = x_ref[...] + a0_ref[...] + a1_ref[...]
    h = jnp.dot(h, W_ref[...], preferred_element_type=jnp.float32) + b_ref[...]
    o_ref[...] = _ln_gelu(h, g_ref[...], be_ref[...])


def _dense2_body(x_ref, a0_ref, a1_ref, W2_ref, b2_ref, g2_ref, be2_ref,
                 Wf_ref, bf_ref, o_ref):
    h = x_ref[...] + a0_ref[...] + a1_ref[...]
    h = jnp.dot(h, W2_ref[...], preferred_element_type=jnp.float32) + b2_ref[...]
    h = _ln_gelu(h, g2_ref[...], be2_ref[...])
    o_ref[...] = jnp.dot(h, Wf_ref[...], preferred_element_type=jnp.float32) + bf_ref[...]


def _row_spec():
    return pl.BlockSpec((_BR, D), lambda i: (i, 0))


def _rep_spec(shape):
    return pl.BlockSpec(shape, lambda i: (0,) * len(shape))


def _dense1(x, a0, a1, W, b, g, be):
    return pl.pallas_call(
        _dense1_body,
        grid=(N // _BR,),
        in_specs=[_row_spec(), _row_spec(), _row_spec(),
                  _rep_spec((D, H)), _rep_spec((1, H)), _rep_spec((1, H)),
                  _rep_spec((1, H))],
        out_specs=_row_spec(),
        out_shape=jax.ShapeDtypeStruct((N, H), jnp.float32),
    )(x, a0, a1, W, b.reshape(1, H), g.reshape(1, H), be.reshape(1, H))


def _dense2(x, a0, a1, W2, b2, g2, be2, Wf, bf):
    return pl.pallas_call(
        _dense2_body,
        grid=(N // _BR,),
        in_specs=[_row_spec(), _row_spec(), _row_spec(),
                  _rep_spec((H, H)), _rep_spec((1, H)), _rep_spec((1, H)),
                  _rep_spec((1, H)),
                  _rep_spec((H, D)), _rep_spec((1, D))],
        out_specs=_row_spec(),
        out_shape=jax.ShapeDtypeStruct((N, D), jnp.float32),
    )(x, a0, a1, W2, b2.reshape(1, H), g2.reshape(1, H), be2.reshape(1, H),
      Wf, bf.reshape(1, D))


def kernel(node_features, adj_indices, adj_values, W1, b1, g1, be1,
           W2, b2, g2, be2, Wf, bf):
    dst2d = adj_indices[0].astype(jnp.int32).reshape(NCHUNK, 1, C)
    src2d = adj_indices[1].astype(jnp.int32).reshape(NCHUNK, 1, C)
    val2d = adj_values.reshape(NCHUNK, 1, C)

    a0, a1 = _sc_aggregate(node_features, dst2d, src2d, val2d)
    x1 = _dense1(node_features, a0, a1, W1, b1, g1, be1)
    c0, c1 = _sc_aggregate(x1, dst2d, src2d, val2d)
    return _dense2(x1, c0, c1, W2, b2, g2, be2, Wf, bf)
